# baseline (speedup 1.0000x reference)
import functools

import jax
import jax.numpy as jnp
import numpy as np
from jax import lax
from jax.experimental import pallas as pl
from jax.experimental.pallas import tpu as pltpu
from jax.experimental.pallas import tpu_sc as plsc

_H = 50
_W = 50
_C = 512
_NA = 9
_N = _H * _W * _NA
_PAD_N = 24576
_ROWS = _PAD_N // 128
_PRE_NMS = 6000
_CAP = 6272
_TRASH = 6144
_CROWS = 48
_POST_NMS = 300
_IOU_T = 0.7
_PW = _W + 4
_PH = _H + 4
_M = 2704

_NEG_INF = float("-inf")


def _anchors_np():
    stride = 16
    yloc = np.arange(stride / 2, 800, stride).astype(int)
    xloc = np.arange(stride / 2, 800, stride).astype(int)
    ctrs = np.array(np.meshgrid(xloc, yloc)).T.reshape(-1, 2)
    sizes = [[stride * s * np.sqrt(r), stride * s * np.sqrt(1.0 / r)]
             for s in (8, 16, 32) for r in (0.5, 1.0, 2.0)]
    anchors = np.empty((0, 4), dtype=np.float32)
    for dim in sizes:
        anchors = np.append(
            anchors,
            np.append(ctrs - np.multiply(0.5, dim),
                      ctrs + np.multiply(0.5, dim), axis=1), axis=0)
    return anchors.astype(np.float32)


def _pad_plane(v):
    out = np.zeros((_PAD_N,), np.float32)
    out[:_N] = v
    return out.reshape(_ROWS, 128)

_ANCH = _anchors_np()
_A0 = _pad_plane(_ANCH[:, 0])
_A1 = _pad_plane(_ANCH[:, 1])
_A2 = _pad_plane(_ANCH[:, 2])
_A3 = _pad_plane(_ANCH[:, 3])


def _trunk_body(x_ref, w1_ref, b1_ref, w2_ref, b2_ref, y_ref):
    acc = jnp.zeros((_M, _C), jnp.float32)
    for t in range(9):
        dy, dx = t // 3, t % 3
        off = dy * _PW + dx
        xs = x_ref[pl.ds(off, _M), :]
        acc = acc + lax.dot_general(
            xs, w1_ref[t], (((1,), (0,)), ((), ())),
            preferred_element_type=jnp.float32)
    rpn = jnp.maximum(acc + b1_ref[0, :][None, :], 0.0)
    y = lax.dot_general(
        rpn, w2_ref[...], (((1,), (0,)), ((), ())),
        preferred_element_type=jnp.float32)
    y_ref[...] = y + b2_ref[0, :][None, :]


def _select_body(l0_ref, l1_ref, ty_ref, tx_ref, th_ref, tw_ref,
                 a0_ref, a1_ref, a2_ref, a3_ref,
                 dest_ref, s_ref, y1_ref, x1_ref, y2_ref, x2_ref):
    shape = (_ROWS, 128)
    row_i = lax.broadcasted_iota(jnp.int32, shape, 0)
    col_i = lax.broadcasted_iota(jnp.int32, shape, 1)
    iota = row_i * 128 + col_i
    valid = iota < _N

    l0 = l0_ref[...]
    l1 = l1_ref[...]
    mx = jnp.maximum(l0, l1)
    e0 = jnp.exp(l0 - mx)
    e1 = jnp.exp(l1 - mx)
    s = e1 / (e0 + e1)

    sbits = lax.bitcast_convert_type(s, jnp.int32)
    sbits = jnp.where(valid, sbits, -1)

    def bs_body(_, lohi):
        lo, hi = lohi
        mid = lo + (hi - lo) // 2
        c = jnp.sum((sbits >= mid).astype(jnp.int32))
        big = c >= _PRE_NMS
        return jnp.where(big, mid, lo), jnp.where(big, hi, mid)

    lo, _ = lax.fori_loop(0, 31, bs_body, (jnp.int32(0), jnp.int32(2**31 - 1)))
    v = lo
    n_gt = jnp.sum((sbits > v).astype(jnp.int32))
    k_tie = _PRE_NMS - n_gt
    tie = sbits == v

    def bs2_body(_, lohi):
        lo2, hi2 = lohi
        mid = lo2 + (hi2 - lo2) // 2
        c = jnp.sum((tie & (iota < mid)).astype(jnp.int32))
        small = c < k_tie
        return jnp.where(small, mid, lo2), jnp.where(small, hi2, mid)

    _, jstar = lax.fori_loop(
        0, 15, bs2_body, (jnp.int32(0), jnp.int32(_PAD_N)))

    eligible = (sbits > v) | (tie & (iota < jstar))

    elig_f = eligible.astype(jnp.float32)
    r128 = lax.broadcasted_iota(jnp.int32, (128, 128), 0)
    c128 = lax.broadcasted_iota(jnp.int32, (128, 128), 1)
    ut128 = (r128 <= c128).astype(jnp.float32)
    incl = lax.dot_general(elig_f, ut128, (((1,), (0,)), ((), ())),
                           preferred_element_type=jnp.float32)
    rowtot = incl[:, 127:128]
    rr = lax.broadcasted_iota(jnp.int32, (_ROWS, _ROWS), 0)
    cc = lax.broadcasted_iota(jnp.int32, (_ROWS, _ROWS), 1)
    lts = (cc < rr).astype(jnp.float32)
    row_excl = lax.dot_general(lts, rowtot, (((1,), (0,)), ((), ())),
                               preferred_element_type=jnp.float32)
    rank = row_excl + incl - elig_f
    dest = jnp.where(eligible, rank.astype(jnp.int32), _TRASH)
    dest_ref[...] = dest
    s_ref[...] = s

    ah = a2_ref[...] - a0_ref[...]
    aw = a3_ref[...] - a1_ref[...]
    acy = a0_ref[...] + 0.5 * ah
    acx = a1_ref[...] + 0.5 * aw
    pcy = ty_ref[...] * ah + acy
    pcx = tx_ref[...] * aw + acx
    ph = jnp.exp(th_ref[...]) * ah
    pw = jnp.exp(tw_ref[...]) * aw
    y1_ref[...] = jnp.clip(pcy - 0.5 * ph, 0.0, 799.0)
    x1_ref[...] = jnp.clip(pcx - 0.5 * pw, 0.0, 799.0)
    y2_ref[...] = jnp.clip(pcy + 0.5 * ph, 0.0, 799.0)
    x2_ref[...] = jnp.clip(pcx + 0.5 * pw, 0.0, 799.0)


def _sc_compact_body(dest_hbm, sv_hbm, y1_hbm, x1_hbm, y2_hbm, x2_hbm,
                     os_hbm, oy1_hbm, ox1_hbm, oy2_hbm, ox2_hbm,
                     dbuf, vb0, vb1, vb2, vb3, vb4, sem):
    wid = lax.axis_index("s") * 2 + lax.axis_index("c")
    pltpu.sync_copy(dest_hbm.at[wid], dbuf)
    ins = (sv_hbm, y1_hbm, x1_hbm, y2_hbm, x2_hbm)
    bufs = (vb0, vb1, vb2, vb3, vb4)
    outs = (os_hbm, oy1_hbm, ox1_hbm, oy2_hbm, ox2_hbm)
    for src, buf in zip(ins, bufs):
        pltpu.sync_copy(src.at[wid], buf)
    handles = []
    for buf, out in zip(bufs, outs):
        for j in range(6):
            handles.append(pltpu.async_copy(buf.at[j], out.at[dbuf.at[j]], sem))
    for h in handles:
        h.wait()


_SC_OUT = jax.ShapeDtypeStruct((_CAP,), jnp.float32)


def _sc_compact(*args):
    fn = functools.partial(
        pl.kernel,
        out_type=[_SC_OUT] * 5,
        mesh=plsc.VectorSubcoreMesh(core_axis_name="c", subcore_axis_name="s"),
        scratch_types=[pltpu.VMEM((6, 128), jnp.int32)]
        + [pltpu.VMEM((6, 128), jnp.float32)] * 5
        + [pltpu.SemaphoreType.DMA],
    )(_sc_compact_body)
    return fn(*args)


def _nms_body(s_ref, y1_ref, x1_ref, y2_ref, x2_ref, out_ref):
    shape = (_CROWS, 128)
    iota = (lax.broadcasted_iota(jnp.int32, shape, 0) * 128
            + lax.broadcasted_iota(jnp.int32, shape, 1))
    live = iota < _PRE_NMS
    s0 = jnp.where(live, s_ref[0:_CROWS, :], _NEG_INF)
    y1 = y1_ref[0:_CROWS, :]
    x1 = x1_ref[0:_CROWS, :]
    y2 = y2_ref[0:_CROWS, :]
    x2 = x2_ref[0:_CROWS, :]
    areas = (y2 - y1) * (x2 - x1)

    oshape = (3, 128)
    rec_i = (lax.broadcasted_iota(jnp.int32, oshape, 0) * 128
             + lax.broadcasted_iota(jnp.int32, oshape, 1))
    zeros3 = jnp.zeros(oshape, jnp.float32)
    fz = jnp.float32(0.0)

    def step(t, carry):
        s, oy1, ox1, oy2, ox2, fy1, fx1, fy2, fx2 = carry
        m = jnp.max(s)
        mask = s == m
        idx = jnp.min(jnp.where(mask, iota, jnp.int32(2**30)))
        oh = iota == idx
        by1 = jnp.sum(jnp.where(oh, y1, 0.0))
        bx1 = jnp.sum(jnp.where(oh, x1, 0.0))
        by2 = jnp.sum(jnp.where(oh, y2, 0.0))
        bx2 = jnp.sum(jnp.where(oh, x2, 0.0))
        first = t == 0
        fy1 = jnp.where(first, by1, fy1)
        fx1 = jnp.where(first, bx1, fx1)
        fy2 = jnp.where(first, by2, fy2)
        fx2 = jnp.where(first, bx2, fx2)
        dead = m == _NEG_INF
        sy1 = jnp.where(dead, fy1, by1)
        sx1 = jnp.where(dead, fx1, bx1)
        sy2 = jnp.where(dead, fy2, by2)
        sx2 = jnp.where(dead, fx2, bx2)
        yy1 = jnp.maximum(sy1, y1)
        xx1 = jnp.maximum(sx1, x1)
        yy2 = jnp.minimum(sy2, y2)
        xx2 = jnp.minimum(sx2, x2)
        inter = jnp.maximum(yy2 - yy1, 0.0) * jnp.maximum(xx2 - xx1, 0.0)
        barea = (sy2 - sy1) * (sx2 - sx1)
        iou = inter / (barea + areas - inter + 1e-9)
        s = jnp.where(iou > _IOU_T, _NEG_INF, s)
        rec = rec_i == t
        oy1 = jnp.where(rec, jnp.floor(sy1), oy1)
        ox1 = jnp.where(rec, jnp.floor(sx1), ox1)
        oy2 = jnp.where(rec, jnp.floor(sy2), oy2)
        ox2 = jnp.where(rec, jnp.floor(sx2), ox2)
        return s, oy1, ox1, oy2, ox2, fy1, fx1, fy2, fx2

    init = (s0, zeros3, zeros3, zeros3, zeros3, fz, fz, fz, fz)
    res = lax.fori_loop(0, _POST_NMS, step, init)
    _, oy1, ox1, oy2, ox2 = res[:5]
    out_ref[...] = jnp.zeros((32, 128), jnp.float32)
    out_ref[0:3, :] = oy1
    out_ref[8:11, :] = ox1
    out_ref[16:19, :] = oy2
    out_ref[24:27, :] = ox2


def kernel(features, W1, b1, Wb, bb, Wc, bc):
    f = jnp.transpose(features[0], (1, 2, 0))
    x = jnp.pad(f, ((1, 3), (1, 3), (0, 0))).reshape(_PH * _PW, _C)
    w1 = jnp.transpose(W1, (2, 3, 1, 0)).reshape(9, _C, _C)
    w2 = jnp.concatenate([Wb[:, :, 0, 0], Wc[:, :, 0, 0]], axis=0)
    w2 = jnp.pad(w2, ((0, 128 - 54), (0, 0))).T
    b2 = jnp.pad(jnp.concatenate([bb, bc]), (0, 128 - 54))

    y = pl.pallas_call(
        _trunk_body,
        out_shape=jax.ShapeDtypeStruct((_M, 128), jnp.float32),
    )(x, w1, b1[None, :], w2, b2[None, :])

    y2 = y[:_H * _PW].reshape(_H, _PW, 128)[:, :_W, :]
    tg_hw = y2[..., :36]
    obj_hw = y2[..., 36:54]
    obj_score = obj_hw.reshape(1, _N, 2)
    tg = tg_hw.reshape(1, _N, 4)

    def plane(vec):
        return jnp.pad(vec, (0, _PAD_N - _N)).reshape(_ROWS, 128)

    obj_f = obj_hw.reshape(_N, 2)
    tg_f = tg_hw.reshape(_N, 4)

    o_dt = jax.ShapeDtypeStruct((_ROWS, 128), jnp.float32)
    dest, sv, by1, bx1, by2, bx2 = pl.pallas_call(
        _select_body,
        out_shape=(jax.ShapeDtypeStruct((_ROWS, 128), jnp.int32),
                   o_dt, o_dt, o_dt, o_dt, o_dt),
    )(plane(obj_f[:, 0]), plane(obj_f[:, 1]),
      plane(tg_f[:, 0]), plane(tg_f[:, 1]),
      plane(tg_f[:, 2]), plane(tg_f[:, 3]),
      jnp.asarray(_A0), jnp.asarray(_A1), jnp.asarray(_A2), jnp.asarray(_A3))

    def chunks(a):
        return a.reshape(32, 6, 128)

    sc, cy1, cx1, cy2, cx2 = _sc_compact(
        chunks(dest), chunks(sv), chunks(by1), chunks(bx1),
        chunks(by2), chunks(bx2))

    outp = pl.pallas_call(
        _nms_body,
        out_shape=jax.ShapeDtypeStruct((32, 128), jnp.float32),
    )(sc.reshape(_CAP // 128, 128), cy1.reshape(_CAP // 128, 128),
      cx1.reshape(_CAP // 128, 128), cy2.reshape(_CAP // 128, 128),
      cx2.reshape(_CAP // 128, 128))

    sel = outp.reshape(4, 8, 128)[:, :3, :].reshape(4, 384)[:, :_POST_NMS]
    props = sel.T[None]
    return props, obj_score, tg


# speedup vs baseline: 10.1694x; 10.1694x over previous
import functools

import jax
import jax.numpy as jnp
import numpy as np
from jax import lax
from jax.experimental import pallas as pl
from jax.experimental.pallas import tpu as pltpu
from jax.experimental.pallas import tpu_sc as plsc

_H = 50
_W = 50
_C = 512
_NA = 9
_N = _H * _W * _NA
_PAD_N = 24576
_ROWS = _PAD_N // 128
_PRE_NMS = 6000
_CAP = 30720
_TRASH = 6144
_CROWS = 48
_POST_NMS = 300
_IOU_T = 0.7
_PW = _W + 4
_PH = _H + 4
_M = 2704

_NEG_INF = float("-inf")


def _anchors_np():
    stride = 16
    yloc = np.arange(stride / 2, 800, stride).astype(int)
    xloc = np.arange(stride / 2, 800, stride).astype(int)
    ctrs = np.array(np.meshgrid(xloc, yloc)).T.reshape(-1, 2)
    sizes = [[stride * s * np.sqrt(r), stride * s * np.sqrt(1.0 / r)]
             for s in (8, 16, 32) for r in (0.5, 1.0, 2.0)]
    anchors = np.empty((0, 4), dtype=np.float32)
    for dim in sizes:
        anchors = np.append(
            anchors,
            np.append(ctrs - np.multiply(0.5, dim),
                      ctrs + np.multiply(0.5, dim), axis=1), axis=0)
    return anchors.astype(np.float32)


def _pad_plane(v):
    out = np.zeros((_PAD_N,), np.float32)
    out[:_N] = v
    return out.reshape(_ROWS, 128)

_ANCH = _anchors_np()
_A0 = _pad_plane(_ANCH[:, 0])
_A1 = _pad_plane(_ANCH[:, 1])
_A2 = _pad_plane(_ANCH[:, 2])
_A3 = _pad_plane(_ANCH[:, 3])


def _trunk_body(x_ref, w1_ref, b1_ref, w2_ref, b2_ref, y_ref):
    acc = jnp.zeros((_M, _C), jnp.float32)
    for t in range(9):
        dy, dx = t // 3, t % 3
        off = dy * _PW + dx
        xs = x_ref[pl.ds(off, _M), :]
        acc = acc + lax.dot_general(
            xs, w1_ref[t], (((1,), (0,)), ((), ())),
            preferred_element_type=jnp.float32)
    rpn = jnp.maximum(acc + b1_ref[0, :][None, :], 0.0)
    y = lax.dot_general(
        rpn, w2_ref[...], (((1,), (0,)), ((), ())),
        preferred_element_type=jnp.float32)
    y_ref[...] = y + b2_ref[0, :][None, :]


def _select_body(l0_ref, l1_ref, ty_ref, tx_ref, th_ref, tw_ref,
                 a0_ref, a1_ref, a2_ref, a3_ref,
                 dest_ref, s_ref, y1_ref, x1_ref, y2_ref, x2_ref):
    shape = (_ROWS, 128)
    row_i = lax.broadcasted_iota(jnp.int32, shape, 0)
    col_i = lax.broadcasted_iota(jnp.int32, shape, 1)
    iota = row_i * 128 + col_i
    valid = iota < _N

    l0 = l0_ref[...]
    l1 = l1_ref[...]
    mx = jnp.maximum(l0, l1)
    e0 = jnp.exp(l0 - mx)
    e1 = jnp.exp(l1 - mx)
    s = e1 / (e0 + e1)

    sbits = lax.bitcast_convert_type(s, jnp.int32)
    sbits = jnp.where(valid, sbits, -1)

    def bs_body(_, lohi):
        lo, hi = lohi
        mid = lo + (hi - lo) // 2
        c = jnp.sum((sbits >= mid).astype(jnp.int32))
        big = c >= _PRE_NMS
        return jnp.where(big, mid, lo), jnp.where(big, hi, mid)

    lo, _ = lax.fori_loop(0, 31, bs_body, (jnp.int32(0), jnp.int32(2**31 - 1)))
    v = lo
    n_gt = jnp.sum((sbits > v).astype(jnp.int32))
    k_tie = _PRE_NMS - n_gt
    tie = sbits == v

    def bs2_body(_, lohi):
        lo2, hi2 = lohi
        mid = lo2 + (hi2 - lo2) // 2
        c = jnp.sum((tie & (iota < mid)).astype(jnp.int32))
        small = c < k_tie
        return jnp.where(small, mid, lo2), jnp.where(small, hi2, mid)

    _, jstar = lax.fori_loop(
        0, 15, bs2_body, (jnp.int32(0), jnp.int32(_PAD_N)))

    eligible = (sbits > v) | (tie & (iota < jstar))

    elig_f = eligible.astype(jnp.float32)
    r128 = lax.broadcasted_iota(jnp.int32, (128, 128), 0)
    c128 = lax.broadcasted_iota(jnp.int32, (128, 128), 1)
    ut128 = (r128 <= c128).astype(jnp.float32)
    incl = lax.dot_general(elig_f, ut128, (((1,), (0,)), ((), ())),
                           preferred_element_type=jnp.float32)
    rowtot = incl[:, 127:128]
    rr = lax.broadcasted_iota(jnp.int32, (_ROWS, _ROWS), 0)
    cc = lax.broadcasted_iota(jnp.int32, (_ROWS, _ROWS), 1)
    lts = (cc < rr).astype(jnp.float32)
    row_excl = lax.dot_general(lts, rowtot, (((1,), (0,)), ((), ())),
                               preferred_element_type=jnp.float32)
    rank = row_excl + incl - elig_f
    irank = iota - rank.astype(jnp.int32)
    dest = jnp.where(eligible, rank.astype(jnp.int32), _TRASH + irank)
    dest_ref[...] = dest
    s_ref[...] = s

    ah = a2_ref[...] - a0_ref[...]
    aw = a3_ref[...] - a1_ref[...]
    acy = a0_ref[...] + 0.5 * ah
    acx = a1_ref[...] + 0.5 * aw
    pcy = ty_ref[...] * ah + acy
    pcx = tx_ref[...] * aw + acx
    ph = jnp.exp(th_ref[...]) * ah
    pw = jnp.exp(tw_ref[...]) * aw
    y1_ref[...] = jnp.clip(pcy - 0.5 * ph, 0.0, 799.0)
    x1_ref[...] = jnp.clip(pcx - 0.5 * pw, 0.0, 799.0)
    y2_ref[...] = jnp.clip(pcy + 0.5 * ph, 0.0, 799.0)
    x2_ref[...] = jnp.clip(pcx + 0.5 * pw, 0.0, 799.0)


def _sc_compact_body(dest_hbm, sv_hbm, y1_hbm, x1_hbm, y2_hbm, x2_hbm,
                     os_hbm, oy1_hbm, ox1_hbm, oy2_hbm, ox2_hbm,
                     dbuf, vb0, vb1, vb2, vb3, vb4, sem):
    wid = lax.axis_index("s") * 2 + lax.axis_index("c")
    pltpu.sync_copy(dest_hbm.at[wid], dbuf)
    ins = (sv_hbm, y1_hbm, x1_hbm, y2_hbm, x2_hbm)
    bufs = (vb0, vb1, vb2, vb3, vb4)
    outs = (os_hbm, oy1_hbm, ox1_hbm, oy2_hbm, ox2_hbm)
    for src, buf in zip(ins, bufs):
        pltpu.sync_copy(src.at[wid], buf)
    handles = []
    for buf, out in zip(bufs, outs):
        for j in range(6):
            handles.append(pltpu.async_copy(buf.at[j], out.at[dbuf.at[j]], sem))
    for h in handles:
        h.wait()


_SC_OUT = jax.ShapeDtypeStruct((_CAP,), jnp.float32)


def _sc_compact(*args):
    fn = functools.partial(
        pl.kernel,
        out_type=[_SC_OUT] * 5,
        mesh=plsc.VectorSubcoreMesh(core_axis_name="c", subcore_axis_name="s"),
        scratch_types=[pltpu.VMEM((6, 128), jnp.int32)]
        + [pltpu.VMEM((6, 128), jnp.float32)] * 5
        + [pltpu.SemaphoreType.DMA],
    )(_sc_compact_body)
    return fn(*args)


def _nms_body(s_ref, y1_ref, x1_ref, y2_ref, x2_ref, out_ref):
    shape = (_CROWS, 128)
    iota = (lax.broadcasted_iota(jnp.int32, shape, 0) * 128
            + lax.broadcasted_iota(jnp.int32, shape, 1))
    live = iota < _PRE_NMS
    s0 = jnp.where(live, s_ref[0:_CROWS, :], _NEG_INF)
    y1 = y1_ref[0:_CROWS, :]
    x1 = x1_ref[0:_CROWS, :]
    y2 = y2_ref[0:_CROWS, :]
    x2 = x2_ref[0:_CROWS, :]
    areas = (y2 - y1) * (x2 - x1)

    oshape = (3, 128)
    rec_i = (lax.broadcasted_iota(jnp.int32, oshape, 0) * 128
             + lax.broadcasted_iota(jnp.int32, oshape, 1))
    zeros3 = jnp.zeros(oshape, jnp.float32)
    fz = jnp.float32(0.0)

    def step(t, carry):
        s, oy1, ox1, oy2, ox2, fy1, fx1, fy2, fx2 = carry
        m = jnp.max(s)
        mask = s == m
        idx = jnp.min(jnp.where(mask, iota, jnp.int32(2**30)))
        oh = iota == idx
        by1 = jnp.sum(jnp.where(oh, y1, 0.0))
        bx1 = jnp.sum(jnp.where(oh, x1, 0.0))
        by2 = jnp.sum(jnp.where(oh, y2, 0.0))
        bx2 = jnp.sum(jnp.where(oh, x2, 0.0))
        first = t == 0
        fy1 = jnp.where(first, by1, fy1)
        fx1 = jnp.where(first, bx1, fx1)
        fy2 = jnp.where(first, by2, fy2)
        fx2 = jnp.where(first, bx2, fx2)
        dead = m == _NEG_INF
        sy1 = jnp.where(dead, fy1, by1)
        sx1 = jnp.where(dead, fx1, bx1)
        sy2 = jnp.where(dead, fy2, by2)
        sx2 = jnp.where(dead, fx2, bx2)
        yy1 = jnp.maximum(sy1, y1)
        xx1 = jnp.maximum(sx1, x1)
        yy2 = jnp.minimum(sy2, y2)
        xx2 = jnp.minimum(sx2, x2)
        inter = jnp.maximum(yy2 - yy1, 0.0) * jnp.maximum(xx2 - xx1, 0.0)
        barea = (sy2 - sy1) * (sx2 - sx1)
        iou = inter / (barea + areas - inter + 1e-9)
        s = jnp.where(iou > _IOU_T, _NEG_INF, s)
        rec = rec_i == t
        oy1 = jnp.where(rec, jnp.floor(sy1), oy1)
        ox1 = jnp.where(rec, jnp.floor(sx1), ox1)
        oy2 = jnp.where(rec, jnp.floor(sy2), oy2)
        ox2 = jnp.where(rec, jnp.floor(sx2), ox2)
        return s, oy1, ox1, oy2, ox2, fy1, fx1, fy2, fx2

    init = (s0, zeros3, zeros3, zeros3, zeros3, fz, fz, fz, fz)
    res = lax.fori_loop(0, _POST_NMS, step, init)
    _, oy1, ox1, oy2, ox2 = res[:5]
    out_ref[...] = jnp.zeros((32, 128), jnp.float32)
    out_ref[0:3, :] = oy1
    out_ref[8:11, :] = ox1
    out_ref[16:19, :] = oy2
    out_ref[24:27, :] = ox2


def kernel(features, W1, b1, Wb, bb, Wc, bc):
    f = jnp.transpose(features[0], (1, 2, 0))
    x = jnp.pad(f, ((1, 3), (1, 3), (0, 0))).reshape(_PH * _PW, _C)
    w1 = jnp.transpose(W1, (2, 3, 1, 0)).reshape(9, _C, _C)
    w2 = jnp.concatenate([Wb[:, :, 0, 0], Wc[:, :, 0, 0]], axis=0)
    w2 = jnp.pad(w2, ((0, 128 - 54), (0, 0))).T
    b2 = jnp.pad(jnp.concatenate([bb, bc]), (0, 128 - 54))

    y = pl.pallas_call(
        _trunk_body,
        out_shape=jax.ShapeDtypeStruct((_M, 128), jnp.float32),
    )(x, w1, b1[None, :], w2, b2[None, :])

    y2 = y[:_H * _PW].reshape(_H, _PW, 128)[:, :_W, :]
    tg_hw = y2[..., :36]
    obj_hw = y2[..., 36:54]
    obj_score = obj_hw.reshape(1, _N, 2)
    tg = tg_hw.reshape(1, _N, 4)

    def plane(vec):
        return jnp.pad(vec, (0, _PAD_N - _N)).reshape(_ROWS, 128)

    obj_f = obj_hw.reshape(_N, 2)
    tg_f = tg_hw.reshape(_N, 4)

    o_dt = jax.ShapeDtypeStruct((_ROWS, 128), jnp.float32)
    dest, sv, by1, bx1, by2, bx2 = pl.pallas_call(
        _select_body,
        out_shape=(jax.ShapeDtypeStruct((_ROWS, 128), jnp.int32),
                   o_dt, o_dt, o_dt, o_dt, o_dt),
    )(plane(obj_f[:, 0]), plane(obj_f[:, 1]),
      plane(tg_f[:, 0]), plane(tg_f[:, 1]),
      plane(tg_f[:, 2]), plane(tg_f[:, 3]),
      jnp.asarray(_A0), jnp.asarray(_A1), jnp.asarray(_A2), jnp.asarray(_A3))

    def chunks(a):
        return a.reshape(32, 6, 128)

    sc, cy1, cx1, cy2, cx2 = _sc_compact(
        chunks(dest), chunks(sv), chunks(by1), chunks(bx1),
        chunks(by2), chunks(bx2))

    outp = pl.pallas_call(
        _nms_body,
        out_shape=jax.ShapeDtypeStruct((32, 128), jnp.float32),
    )(sc.reshape(_CAP // 128, 128), cy1.reshape(_CAP // 128, 128),
      cx1.reshape(_CAP // 128, 128), cy2.reshape(_CAP // 128, 128),
      cx2.reshape(_CAP // 128, 128))

    sel = outp.reshape(4, 8, 128)[:, :3, :].reshape(4, 384)[:, :_POST_NMS]
    props = sel.T[None]
    return props, obj_score, tg


# speedup vs baseline: 18.6966x; 1.8385x over previous
import jax
import jax.numpy as jnp
import numpy as np
from jax import lax
from jax.experimental import pallas as pl
from jax.experimental.pallas import tpu as pltpu

_H = 50
_W = 50
_C = 512
_NA = 9
_N = _H * _W * _NA
_PAD_N = 22528
_ROWS = _PAD_N // 128
_PRE_NMS = 6000
_POST_NMS = 300
_IOU_T = 0.7
_PW = _W + 4
_PH = _H + 4
_M = 2704

_NEG_INF = float("-inf")


def _anchors_np():
    stride = 16
    yloc = np.arange(stride / 2, 800, stride).astype(int)
    xloc = np.arange(stride / 2, 800, stride).astype(int)
    ctrs = np.array(np.meshgrid(xloc, yloc)).T.reshape(-1, 2)
    sizes = [[stride * s * np.sqrt(r), stride * s * np.sqrt(1.0 / r)]
             for s in (8, 16, 32) for r in (0.5, 1.0, 2.0)]
    anchors = np.empty((0, 4), dtype=np.float32)
    for dim in sizes:
        anchors = np.append(
            anchors,
            np.append(ctrs - np.multiply(0.5, dim),
                      ctrs + np.multiply(0.5, dim), axis=1), axis=0)
    return anchors.astype(np.float32)


def _pad_plane(v):
    out = np.zeros((_PAD_N,), np.float32)
    out[:_N] = v
    return out.reshape(_ROWS, 128)

_ANCH = _anchors_np()
_A0 = _pad_plane(_ANCH[:, 0])
_A1 = _pad_plane(_ANCH[:, 1])
_A2 = _pad_plane(_ANCH[:, 2])
_A3 = _pad_plane(_ANCH[:, 3])


def _trunk_body(x_ref, w1_ref, b1_ref, w2_ref, b2_ref, y_ref):
    acc = jnp.zeros((_M, _C), jnp.float32)
    for t in range(9):
        dy, dx = t // 3, t % 3
        off = dy * _PW + dx
        xs = x_ref[pl.ds(off, _M), :]
        acc = acc + lax.dot_general(
            xs, w1_ref[t], (((1,), (0,)), ((), ())),
            preferred_element_type=jnp.float32)
    rpn = jnp.maximum(acc + b1_ref[0, :][None, :], 0.0)
    y = lax.dot_general(
        rpn, w2_ref[...], (((1,), (0,)), ((), ())),
        preferred_element_type=jnp.float32)
    y_ref[...] = y + b2_ref[0, :][None, :]


def _nms_body(l0_ref, l1_ref, ty_ref, tx_ref, th_ref, tw_ref,
              a0_ref, a1_ref, a2_ref, a3_ref, out_ref,
              y1s, x1s, y2s, x2s, ars):
    shape = (_ROWS, 128)
    row_i = lax.broadcasted_iota(jnp.int32, shape, 0)
    col_i = lax.broadcasted_iota(jnp.int32, shape, 1)
    iota = row_i * 128 + col_i
    valid = iota < _N

    l0 = l0_ref[...]
    l1 = l1_ref[...]
    mx = jnp.maximum(l0, l1)
    e0 = jnp.exp(l0 - mx)
    e1 = jnp.exp(l1 - mx)
    s = e1 / (e0 + e1)

    sbits = lax.bitcast_convert_type(s, jnp.int32)
    sbits = jnp.where(valid, sbits, -1)

    def bs_body(_, lohi):
        lo, hi = lohi
        mid = lo + (hi - lo) // 2
        c = jnp.sum((sbits >= mid).astype(jnp.int32))
        big = c >= _PRE_NMS
        return jnp.where(big, mid, lo), jnp.where(big, hi, mid)

    lo, _ = lax.fori_loop(0, 31, bs_body, (jnp.int32(0), jnp.int32(2**31 - 1)))
    v = lo
    n_gt = jnp.sum((sbits > v).astype(jnp.int32))
    k_tie = _PRE_NMS - n_gt
    tie = sbits == v

    def bs2_body(_, lohi):
        lo2, hi2 = lohi
        mid = lo2 + (hi2 - lo2) // 2
        c = jnp.sum((tie & (iota < mid)).astype(jnp.int32))
        small = c < k_tie
        return jnp.where(small, mid, lo2), jnp.where(small, hi2, mid)

    _, jstar = lax.fori_loop(
        0, 15, bs2_body, (jnp.int32(0), jnp.int32(_PAD_N)))

    eligible = (sbits > v) | (tie & (iota < jstar))
    s_nms = jnp.where(eligible, s, _NEG_INF)

    ah = a2_ref[...] - a0_ref[...]
    aw = a3_ref[...] - a1_ref[...]
    acy = a0_ref[...] + 0.5 * ah
    acx = a1_ref[...] + 0.5 * aw
    pcy = ty_ref[...] * ah + acy
    pcx = tx_ref[...] * aw + acx
    ph = jnp.exp(th_ref[...]) * ah
    pw = jnp.exp(tw_ref[...]) * aw
    y1 = jnp.clip(pcy - 0.5 * ph, 0.0, 799.0)
    x1 = jnp.clip(pcx - 0.5 * pw, 0.0, 799.0)
    y2 = jnp.clip(pcy + 0.5 * ph, 0.0, 799.0)
    x2 = jnp.clip(pcx + 0.5 * pw, 0.0, 799.0)
    y1s[...] = y1
    x1s[...] = x1
    y2s[...] = y2
    x2s[...] = x2
    ars[...] = (y2 - y1) * (x2 - x1)

    oshape = (3, 128)
    rec_i = (lax.broadcasted_iota(jnp.int32, oshape, 0) * 128
             + lax.broadcasted_iota(jnp.int32, oshape, 1))
    lane_i = lax.broadcasted_iota(jnp.int32, (1, 128), 1)
    zeros3 = jnp.zeros(oshape, jnp.float32)
    fz = jnp.float32(0.0)

    def step(t, carry):
        s, oy1, ox1, oy2, ox2, fy1, fx1, fy2, fx2 = carry
        m = jnp.max(s)
        mask = s == m
        idx = jnp.min(jnp.where(mask, iota, jnp.int32(2**30)))
        r = idx >> 7
        lane_oh = lane_i == (idx & 127)
        by1 = jnp.sum(jnp.where(lane_oh, y1s[pl.ds(r, 1), :], 0.0))
        bx1 = jnp.sum(jnp.where(lane_oh, x1s[pl.ds(r, 1), :], 0.0))
        by2 = jnp.sum(jnp.where(lane_oh, y2s[pl.ds(r, 1), :], 0.0))
        bx2 = jnp.sum(jnp.where(lane_oh, x2s[pl.ds(r, 1), :], 0.0))
        first = t == 0
        fy1 = jnp.where(first, by1, fy1)
        fx1 = jnp.where(first, bx1, fx1)
        fy2 = jnp.where(first, by2, fy2)
        fx2 = jnp.where(first, bx2, fx2)
        dead = m == _NEG_INF
        sy1 = jnp.where(dead, fy1, by1)
        sx1 = jnp.where(dead, fx1, bx1)
        sy2 = jnp.where(dead, fy2, by2)
        sx2 = jnp.where(dead, fx2, bx2)
        y1f = y1s[...]
        x1f = x1s[...]
        y2f = y2s[...]
        x2f = x2s[...]
        yy1 = jnp.maximum(sy1, y1f)
        xx1 = jnp.maximum(sx1, x1f)
        yy2 = jnp.minimum(sy2, y2f)
        xx2 = jnp.minimum(sx2, x2f)
        inter = jnp.maximum(yy2 - yy1, 0.0) * jnp.maximum(xx2 - xx1, 0.0)
        barea = (sy2 - sy1) * (sx2 - sx1)
        iou = inter / (barea + ars[...] - inter + 1e-9)
        s = jnp.where(iou > _IOU_T, _NEG_INF, s)
        rec = rec_i == t
        oy1 = jnp.where(rec, jnp.floor(sy1), oy1)
        ox1 = jnp.where(rec, jnp.floor(sx1), ox1)
        oy2 = jnp.where(rec, jnp.floor(sy2), oy2)
        ox2 = jnp.where(rec, jnp.floor(sx2), ox2)
        return s, oy1, ox1, oy2, ox2, fy1, fx1, fy2, fx2

    init = (s_nms, zeros3, zeros3, zeros3, zeros3, fz, fz, fz, fz)
    res = lax.fori_loop(0, _POST_NMS, step, init)
    _, oy1, ox1, oy2, ox2 = res[:5]
    out_ref[...] = jnp.zeros((32, 128), jnp.float32)
    out_ref[0:3, :] = oy1
    out_ref[8:11, :] = ox1
    out_ref[16:19, :] = oy2
    out_ref[24:27, :] = ox2


def kernel(features, W1, b1, Wb, bb, Wc, bc):
    f = jnp.transpose(features[0], (1, 2, 0))
    x = jnp.pad(f, ((1, 3), (1, 3), (0, 0))).reshape(_PH * _PW, _C)
    w1 = jnp.transpose(W1, (2, 3, 1, 0)).reshape(9, _C, _C)
    w2 = jnp.concatenate([Wb[:, :, 0, 0], Wc[:, :, 0, 0]], axis=0)
    w2 = jnp.pad(w2, ((0, 128 - 54), (0, 0))).T
    b2 = jnp.pad(jnp.concatenate([bb, bc]), (0, 128 - 54))

    y = pl.pallas_call(
        _trunk_body,
        out_shape=jax.ShapeDtypeStruct((_M, 128), jnp.float32),
    )(x, w1, b1[None, :], w2, b2[None, :])

    y2 = y[:_H * _PW].reshape(_H, _PW, 128)[:, :_W, :]
    tg_hw = y2[..., :36]
    obj_hw = y2[..., 36:54]
    obj_score = obj_hw.reshape(1, _N, 2)
    tg = tg_hw.reshape(1, _N, 4)

    def plane(vec):
        return jnp.pad(vec, (0, _PAD_N - _N)).reshape(_ROWS, 128)

    obj_f = obj_hw.reshape(_N, 2)
    tg_f = tg_hw.reshape(_N, 4)

    outp = pl.pallas_call(
        _nms_body,
        out_shape=jax.ShapeDtypeStruct((32, 128), jnp.float32),
        scratch_shapes=[pltpu.VMEM((_ROWS, 128), jnp.float32)] * 5,
    )(plane(obj_f[:, 0]), plane(obj_f[:, 1]),
      plane(tg_f[:, 0]), plane(tg_f[:, 1]),
      plane(tg_f[:, 2]), plane(tg_f[:, 3]),
      jnp.asarray(_A0), jnp.asarray(_A1), jnp.asarray(_A2), jnp.asarray(_A3))

    sel = outp.reshape(4, 8, 128)[:, :3, :].reshape(4, 384)[:, :_POST_NMS]
    props = sel.T[None]
    return props, obj_score, tg
